# Initial kernel scaffold; baseline (speedup 1.0000x reference)
import jax
import jax.numpy as jnp
from jax.experimental import pallas as pl


def kernel(e_rbf, z, nbr_list, W_rbf, emb_table, W_edge, b_edge):
    raise NotImplementedError("write your pallas kernel here")


# speedup vs baseline: 2.4407x; 2.4407x over previous
import functools

import jax
import jax.numpy as jnp
from jax import lax
from jax.experimental import pallas as pl
from jax.experimental.pallas import tpu as pltpu
from jax.experimental.pallas import tpu_sc as plsc

N_NODES = 10000
N_EDGES = 320000
N_RBF = 16
EMBED_DIM = 128


_NC, _NS, _L = 2, 16, 16
_NW = _NC * _NS
_TOTAL = 2 * N_EDGES
_PER_W = _TOTAL // _NW


def _sc_gather_body(z_hbm, nbr_hbm, out_hbm, z_v, idx_v, out_v):
    wid = lax.axis_index("s") * _NC + lax.axis_index("c")
    base = wid * _PER_W
    pltpu.sync_copy(z_hbm, z_v)
    pltpu.sync_copy(nbr_hbm.at[pl.ds(base, _PER_W)], idx_v)

    def body(k, carry):
        sl = pl.ds(k * _L, _L)
        out_v[sl] = plsc.load_gather(z_v, [idx_v[sl]])
        return carry

    lax.fori_loop(0, _PER_W // _L, body, 0, unroll=8)
    pltpu.sync_copy(out_v, out_hbm.at[pl.ds(base, _PER_W)])


@jax.jit
def _sc_index_gather(z_i32, nbr_flat):
    mesh = plsc.VectorSubcoreMesh(core_axis_name="c", subcore_axis_name="s")
    return pl.kernel(
        _sc_gather_body,
        out_type=jax.ShapeDtypeStruct((_TOTAL,), jnp.int32),
        mesh=mesh,
        scratch_types=[
            pltpu.VMEM((N_NODES,), jnp.int32),
            pltpu.VMEM((_PER_W,), jnp.int32),
            pltpu.VMEM((_PER_W,), jnp.int32),
        ],
        compiler_params=pltpu.CompilerParams(needs_layout_passes=False),
    )(z_i32, nbr_flat)


_BLK = 2000


def _tc_dense_body(sd_ref, e_ref, t1_ref, t2_ref, wc_ref, b_ref, o_ref):
    sd = sd_ref[...]
    cols = lax.broadcasted_iota(jnp.int32, (_BLK, EMBED_DIM), 1)
    oh_s = (cols == sd[:, 0:1]).astype(jnp.float32)
    oh_d = (cols == sd[:, 1:2]).astype(jnp.float32)
    acc = jnp.dot(oh_s, t1_ref[...], preferred_element_type=jnp.float32)
    acc = acc + jnp.dot(oh_d, t2_ref[...], preferred_element_type=jnp.float32)
    acc = acc + jnp.dot(e_ref[...], wc_ref[...], preferred_element_type=jnp.float32)
    acc = acc + b_ref[...]
    o_ref[...] = acc * (1.0 / (1.0 + jnp.exp(-acc)))


@jax.jit
def _tc_dense(sd, e_rbf, t1, t2, w_c, b):
    grid = (N_EDGES // _BLK,)
    return pl.pallas_call(
        _tc_dense_body,
        grid=grid,
        in_specs=[
            pl.BlockSpec((_BLK, 2), lambda i: (i, 0)),
            pl.BlockSpec((_BLK, N_RBF), lambda i: (i, 0)),
            pl.BlockSpec((EMBED_DIM, EMBED_DIM), lambda i: (0, 0)),
            pl.BlockSpec((EMBED_DIM, EMBED_DIM), lambda i: (0, 0)),
            pl.BlockSpec((N_RBF, EMBED_DIM), lambda i: (0, 0)),
            pl.BlockSpec((1, EMBED_DIM), lambda i: (0, 0)),
        ],
        out_specs=pl.BlockSpec((_BLK, EMBED_DIM), lambda i: (i, 0)),
        out_shape=jax.ShapeDtypeStruct((N_EDGES, EMBED_DIM), jnp.float32),
        compiler_params=pltpu.CompilerParams(
            dimension_semantics=("arbitrary",),
        ),
    )(sd, e_rbf, t1, t2, w_c, b)


def kernel(e_rbf, z, nbr_list, W_rbf, emb_table, W_edge, b_edge):
    W1 = W_edge[:EMBED_DIM]
    W2 = W_edge[EMBED_DIM : 2 * EMBED_DIM]
    W3 = W_edge[2 * EMBED_DIM :]
    t1 = jnp.zeros((EMBED_DIM, EMBED_DIM), jnp.float32).at[:100].set(emb_table @ W1)
    t2 = jnp.zeros((EMBED_DIM, EMBED_DIM), jnp.float32).at[:100].set(emb_table @ W2)
    w_c = W_rbf @ W3
    b = b_edge.reshape(1, EMBED_DIM)

    z_i32 = z.astype(jnp.int32)
    nbr_flat = nbr_list.reshape(-1)
    sd = _sc_index_gather(z_i32, nbr_flat).reshape(N_EDGES, 2)
    return _tc_dense(sd, e_rbf, t1, t2, w_c, b)


# speedup vs baseline: 3.4723x; 1.4227x over previous
import functools

import jax
import jax.numpy as jnp
from jax import lax
from jax.experimental import pallas as pl
from jax.experimental.pallas import tpu as pltpu
from jax.experimental.pallas import tpu_sc as plsc

N_NODES = 10000
N_EDGES = 320000
N_RBF = 16
EMBED_DIM = 128


_NC, _NS, _L = 2, 16, 16
_NW = _NC * _NS
_EPW = N_EDGES // _NW


def _sc_gather_body(z_hbm, nbr_hbm, out_hbm, z_v, idx_v, s_v, d_v):
    wid = lax.axis_index("s") * _NC + lax.axis_index("c")
    ebase = wid * _EPW
    pltpu.sync_copy(z_hbm, z_v)
    pltpu.sync_copy(nbr_hbm.at[pl.ds(2 * ebase, 2 * _EPW)], idx_v)
    lane = lax.iota(jnp.int32, _L)

    def body(k, carry):
        src = plsc.load_gather(idx_v, [k * (2 * _L) + 2 * lane])
        dst = plsc.load_gather(idx_v, [k * (2 * _L) + 2 * lane + 1])
        sl = pl.ds(k * _L, _L)
        s_v[sl] = plsc.load_gather(z_v, [src])
        d_v[sl] = plsc.load_gather(z_v, [dst])
        return carry

    lax.fori_loop(0, _EPW // _L, body, 0, unroll=8)
    pltpu.sync_copy(s_v, out_hbm.at[pl.ds(ebase, _EPW)])
    pltpu.sync_copy(d_v, out_hbm.at[pl.ds(N_EDGES + ebase, _EPW)])


@jax.jit
def _sc_index_gather(z_i32, nbr_flat):
    mesh = plsc.VectorSubcoreMesh(core_axis_name="c", subcore_axis_name="s")
    return pl.kernel(
        _sc_gather_body,
        out_type=jax.ShapeDtypeStruct((2 * N_EDGES,), jnp.int32),
        mesh=mesh,
        scratch_types=[
            pltpu.VMEM((N_NODES,), jnp.int32),
            pltpu.VMEM((2 * _EPW,), jnp.int32),
            pltpu.VMEM((_EPW,), jnp.int32),
            pltpu.VMEM((_EPW,), jnp.int32),
        ],
        compiler_params=pltpu.CompilerParams(needs_layout_passes=False),
    )(z_i32, nbr_flat)


_BLK = 2000


_DOTT = (((0,), (0,)), ((), ()))


def _tc_dense_body(s_ref, d_ref, e_ref, t1_ref, t2_ref, wc_ref, b_ref, o_ref):
    s = s_ref[0]
    d = d_ref[0]
    rows = lax.broadcasted_iota(jnp.int32, (EMBED_DIM, _BLK), 0)
    ohT_s = (rows == s).astype(jnp.float32)
    ohT_d = (rows == d).astype(jnp.float32)
    acc = lax.dot_general(ohT_s, t1_ref[...], _DOTT,
                          preferred_element_type=jnp.float32)
    acc = acc + lax.dot_general(ohT_d, t2_ref[...], _DOTT,
                                preferred_element_type=jnp.float32)
    acc = acc + jnp.dot(e_ref[...], wc_ref[...], preferred_element_type=jnp.float32)
    acc = acc + b_ref[...]
    o_ref[...] = acc * (1.0 / (1.0 + jnp.exp(-acc)))


@jax.jit
def _tc_dense(s3, d3, e_rbf, t1, t2, w_c, b):
    grid = (N_EDGES // _BLK,)
    return pl.pallas_call(
        _tc_dense_body,
        grid=grid,
        in_specs=[
            pl.BlockSpec((1, 1, _BLK), lambda i: (i, 0, 0)),
            pl.BlockSpec((1, 1, _BLK), lambda i: (i, 0, 0)),
            pl.BlockSpec((_BLK, N_RBF), lambda i: (i, 0)),
            pl.BlockSpec((EMBED_DIM, EMBED_DIM), lambda i: (0, 0)),
            pl.BlockSpec((EMBED_DIM, EMBED_DIM), lambda i: (0, 0)),
            pl.BlockSpec((N_RBF, EMBED_DIM), lambda i: (0, 0)),
            pl.BlockSpec((1, EMBED_DIM), lambda i: (0, 0)),
        ],
        out_specs=pl.BlockSpec((_BLK, EMBED_DIM), lambda i: (i, 0)),
        out_shape=jax.ShapeDtypeStruct((N_EDGES, EMBED_DIM), jnp.float32),
        compiler_params=pltpu.CompilerParams(
            dimension_semantics=("arbitrary",),
        ),
    )(s3, d3, e_rbf, t1, t2, w_c, b)


def kernel(e_rbf, z, nbr_list, W_rbf, emb_table, W_edge, b_edge):
    W1 = W_edge[:EMBED_DIM]
    W2 = W_edge[EMBED_DIM : 2 * EMBED_DIM]
    W3 = W_edge[2 * EMBED_DIM :]
    t1 = jnp.zeros((EMBED_DIM, EMBED_DIM), jnp.float32).at[:100].set(emb_table @ W1)
    t2 = jnp.zeros((EMBED_DIM, EMBED_DIM), jnp.float32).at[:100].set(emb_table @ W2)
    w_c = W_rbf @ W3
    b = b_edge.reshape(1, EMBED_DIM)

    z_i32 = z.astype(jnp.int32)
    nbr_flat = nbr_list.reshape(-1)
    sd = _sc_index_gather(z_i32, nbr_flat)
    nblk = N_EDGES // _BLK
    s3 = sd[:N_EDGES].reshape(nblk, 1, _BLK)
    d3 = sd[N_EDGES:].reshape(nblk, 1, _BLK)
    return _tc_dense(s3, d3, e_rbf, t1, t2, w_c, b)


# speedup vs baseline: 4.0301x; 1.1606x over previous
import functools

import jax
import jax.numpy as jnp
from jax import lax
from jax.experimental import pallas as pl
from jax.experimental.pallas import tpu as pltpu
from jax.experimental.pallas import tpu_sc as plsc

N_NODES = 10000
N_EDGES = 320000
N_RBF = 16
EMBED_DIM = 128


_NC, _NS, _L = 2, 16, 16
_NW = _NC * _NS
_EPW = N_EDGES // _NW


def _sc_gather_body(z_hbm, nbr_hbm, out_hbm, z_v, idx_v, s_v, d_v):
    wid = lax.axis_index("s") * _NC + lax.axis_index("c")
    ebase = wid * _EPW
    pltpu.sync_copy(z_hbm, z_v)
    pltpu.sync_copy(nbr_hbm.at[pl.ds(2 * ebase, 2 * _EPW)], idx_v)
    lane = lax.iota(jnp.int32, _L)

    def body(k, carry):
        flat = k * (2 * _L) + 2 * lane
        src = plsc.load_gather(idx_v, [flat])
        dst = plsc.load_gather(idx_v, [flat + 1])
        sl = pl.ds(k * _L, _L)
        s_v[sl] = plsc.load_gather(z_v, [src])
        d_v[sl] = plsc.load_gather(z_v, [dst])
        return carry

    lax.fori_loop(0, _EPW // _L, body, 0, unroll=8)
    pltpu.sync_copy(s_v, out_hbm.at[pl.ds(ebase, _EPW)])
    pltpu.sync_copy(d_v, out_hbm.at[pl.ds(N_EDGES + ebase, _EPW)])


@jax.jit
def _sc_index_gather(z_i32, nbr_flat):
    mesh = plsc.VectorSubcoreMesh(core_axis_name="c", subcore_axis_name="s")
    return pl.kernel(
        _sc_gather_body,
        out_type=jax.ShapeDtypeStruct((2 * N_EDGES,), jnp.int32),
        mesh=mesh,
        scratch_types=[
            pltpu.VMEM((N_NODES,), jnp.int32),
            pltpu.VMEM((2 * _EPW,), jnp.int32),
            pltpu.VMEM((_EPW,), jnp.int32),
            pltpu.VMEM((_EPW,), jnp.int32),
        ],
        compiler_params=pltpu.CompilerParams(needs_layout_passes=False),
    )(z_i32, nbr_flat)


_BLK = 8000


_DOTT = (((0,), (0,)), ((), ()))


def _tc_dense_body(s_ref, d_ref, e_ref, t1_ref, t2_ref, wc_ref, b_ref, o_ref):
    s = s_ref[0]
    d = d_ref[0]
    rows = lax.broadcasted_iota(jnp.int32, (EMBED_DIM, _BLK), 0)
    ohT_s = (rows == s).astype(jnp.float32)
    ohT_d = (rows == d).astype(jnp.float32)
    acc = lax.dot_general(ohT_s, t1_ref[...], _DOTT,
                          preferred_element_type=jnp.float32)
    acc = acc + lax.dot_general(ohT_d, t2_ref[...], _DOTT,
                                preferred_element_type=jnp.float32)
    acc = acc + jnp.dot(e_ref[...], wc_ref[...], preferred_element_type=jnp.float32)
    acc = acc + b_ref[...]
    o_ref[...] = acc * (1.0 / (1.0 + jnp.exp(-acc)))


@jax.jit
def _tc_dense(s3, d3, e_rbf, t1, t2, w_c, b):
    grid = (N_EDGES // _BLK,)
    return pl.pallas_call(
        _tc_dense_body,
        grid=grid,
        in_specs=[
            pl.BlockSpec((1, 1, _BLK), lambda i: (i, 0, 0)),
            pl.BlockSpec((1, 1, _BLK), lambda i: (i, 0, 0)),
            pl.BlockSpec((_BLK, N_RBF), lambda i: (i, 0)),
            pl.BlockSpec((EMBED_DIM, EMBED_DIM), lambda i: (0, 0)),
            pl.BlockSpec((EMBED_DIM, EMBED_DIM), lambda i: (0, 0)),
            pl.BlockSpec((N_RBF, EMBED_DIM), lambda i: (0, 0)),
            pl.BlockSpec((1, EMBED_DIM), lambda i: (0, 0)),
        ],
        out_specs=pl.BlockSpec((_BLK, EMBED_DIM), lambda i: (i, 0)),
        out_shape=jax.ShapeDtypeStruct((N_EDGES, EMBED_DIM), jnp.float32),
        compiler_params=pltpu.CompilerParams(
            dimension_semantics=("arbitrary",),
        ),
    )(s3, d3, e_rbf, t1, t2, w_c, b)


def kernel(e_rbf, z, nbr_list, W_rbf, emb_table, W_edge, b_edge):
    W1 = W_edge[:EMBED_DIM]
    W2 = W_edge[EMBED_DIM : 2 * EMBED_DIM]
    W3 = W_edge[2 * EMBED_DIM :]
    t1 = jnp.zeros((EMBED_DIM, EMBED_DIM), jnp.float32).at[:100].set(emb_table @ W1)
    t2 = jnp.zeros((EMBED_DIM, EMBED_DIM), jnp.float32).at[:100].set(emb_table @ W2)
    w_c = W_rbf @ W3
    b = b_edge.reshape(1, EMBED_DIM)

    z_i32 = z.astype(jnp.int32)
    sd = _sc_index_gather(z_i32, nbr_list.reshape(-1))
    nblk = N_EDGES // _BLK
    s3 = sd[:N_EDGES].reshape(nblk, 1, _BLK)
    d3 = sd[N_EDGES:].reshape(nblk, 1, _BLK)
    return _tc_dense(s3, d3, e_rbf, t1, t2, w_c, b)


# speedup vs baseline: 4.1705x; 1.0348x over previous
import functools

import jax
import jax.numpy as jnp
from jax import lax
from jax.experimental import pallas as pl
from jax.experimental.pallas import tpu as pltpu
from jax.experimental.pallas import tpu_sc as plsc

N_NODES = 10000
N_EDGES = 320000
N_RBF = 16
EMBED_DIM = 128


_NC, _NS, _L = 2, 16, 16
_NW = _NC * _NS
_EPW = N_EDGES // _NW


def _sc_gather_body(z_hbm, nbr_hbm, out_hbm, z_v, idx_v, s_v, d_v):
    wid = lax.axis_index("s") * _NC + lax.axis_index("c")
    ebase = wid * _EPW
    pltpu.sync_copy(z_hbm, z_v)
    pltpu.sync_copy(nbr_hbm.at[pl.ds(2 * ebase, 2 * _EPW)], idx_v)
    lane = lax.iota(jnp.int32, _L)

    def body(k, carry):
        flat = k * (2 * _L) + 2 * lane
        src = plsc.load_gather(idx_v, [flat])
        dst = plsc.load_gather(idx_v, [flat + 1])
        sl = pl.ds(k * _L, _L)
        s_v[sl] = plsc.load_gather(z_v, [src])
        d_v[sl] = plsc.load_gather(z_v, [dst])
        return carry

    lax.fori_loop(0, _EPW // _L, body, 0, unroll=8)
    pltpu.sync_copy(s_v, out_hbm.at[pl.ds(ebase, _EPW)])
    pltpu.sync_copy(d_v, out_hbm.at[pl.ds(N_EDGES + ebase, _EPW)])


@jax.jit
def _sc_index_gather(z_i32, nbr_flat):
    mesh = plsc.VectorSubcoreMesh(core_axis_name="c", subcore_axis_name="s")
    return pl.kernel(
        _sc_gather_body,
        out_type=jax.ShapeDtypeStruct((2 * N_EDGES,), jnp.int32),
        mesh=mesh,
        scratch_types=[
            pltpu.VMEM((N_NODES,), jnp.int32),
            pltpu.VMEM((2 * _EPW,), jnp.int32),
            pltpu.VMEM((_EPW,), jnp.int32),
            pltpu.VMEM((_EPW,), jnp.int32),
        ],
        compiler_params=pltpu.CompilerParams(needs_layout_passes=False),
    )(z_i32, nbr_flat)


_BLK = 16000


_DOTT = (((0,), (0,)), ((), ()))


def _tc_dense_body(s_ref, d_ref, e_ref, t1_ref, t2_ref, wc_ref, b_ref, o_ref):
    s = s_ref[0]
    d = d_ref[0]
    rows = lax.broadcasted_iota(jnp.int32, (EMBED_DIM, _BLK), 0)
    ohT_s = (rows == s).astype(jnp.float32)
    ohT_d = (rows == d).astype(jnp.float32)
    acc = lax.dot_general(ohT_s, t1_ref[...], _DOTT,
                          preferred_element_type=jnp.float32)
    acc = acc + lax.dot_general(ohT_d, t2_ref[...], _DOTT,
                                preferred_element_type=jnp.float32)
    acc = acc + jnp.dot(e_ref[...], wc_ref[...], preferred_element_type=jnp.float32)
    acc = acc + b_ref[...]
    o_ref[...] = acc * (1.0 / (1.0 + jnp.exp(-acc)))


@jax.jit
def _tc_dense(s3, d3, e_rbf, t1, t2, w_c, b):
    grid = (N_EDGES // _BLK,)
    return pl.pallas_call(
        _tc_dense_body,
        grid=grid,
        in_specs=[
            pl.BlockSpec((1, 1, _BLK), lambda i: (i, 0, 0)),
            pl.BlockSpec((1, 1, _BLK), lambda i: (i, 0, 0)),
            pl.BlockSpec((_BLK, N_RBF), lambda i: (i, 0)),
            pl.BlockSpec((EMBED_DIM, EMBED_DIM), lambda i: (0, 0)),
            pl.BlockSpec((EMBED_DIM, EMBED_DIM), lambda i: (0, 0)),
            pl.BlockSpec((N_RBF, EMBED_DIM), lambda i: (0, 0)),
            pl.BlockSpec((1, EMBED_DIM), lambda i: (0, 0)),
        ],
        out_specs=pl.BlockSpec((_BLK, EMBED_DIM), lambda i: (i, 0)),
        out_shape=jax.ShapeDtypeStruct((N_EDGES, EMBED_DIM), jnp.float32),
        compiler_params=pltpu.CompilerParams(
            dimension_semantics=("arbitrary",),
        ),
    )(s3, d3, e_rbf, t1, t2, w_c, b)


def kernel(e_rbf, z, nbr_list, W_rbf, emb_table, W_edge, b_edge):
    W1 = W_edge[:EMBED_DIM]
    W2 = W_edge[EMBED_DIM : 2 * EMBED_DIM]
    W3 = W_edge[2 * EMBED_DIM :]
    t1 = jnp.zeros((EMBED_DIM, EMBED_DIM), jnp.float32).at[:100].set(emb_table @ W1)
    t2 = jnp.zeros((EMBED_DIM, EMBED_DIM), jnp.float32).at[:100].set(emb_table @ W2)
    w_c = W_rbf @ W3
    b = b_edge.reshape(1, EMBED_DIM)

    z_i32 = z.astype(jnp.int32)
    sd = _sc_index_gather(z_i32, nbr_list.reshape(-1))
    nblk = N_EDGES // _BLK
    s3 = sd[:N_EDGES].reshape(nblk, 1, _BLK)
    d3 = sd[N_EDGES:].reshape(nblk, 1, _BLK)
    return _tc_dense(s3, d3, e_rbf, t1, t2, w_c, b)


# speedup vs baseline: 5.1627x; 1.2379x over previous
import functools

import jax
import jax.numpy as jnp
from jax import lax
from jax.experimental import pallas as pl
from jax.experimental.pallas import tpu as pltpu
from jax.experimental.pallas import tpu_sc as plsc

N_NODES = 10000
N_EDGES = 320000
N_RBF = 16
EMBED_DIM = 128


_NC, _NS, _L = 2, 16, 16
_NW = _NC * _NS
_EPW = N_EDGES // _NW


_TPW = _EPW // 8
_CT = 50
_NCHUNK = _TPW // _CT


def _sc_gather_body(z_hbm, nbr_hbm, out_hbm, z_v, nb_v, s_v, d_v):
    wid = lax.axis_index("s") * _NC + lax.axis_index("c")
    ebase = wid * _EPW
    tbase = wid * _TPW
    pltpu.sync_copy(z_hbm, z_v)
    lane = lax.iota(jnp.int32, _L)
    t_off = lax.shift_right_logical(lane, 1 + 2)
    r_idx = lane & 7
    zero = jnp.zeros((_L,), jnp.int32)
    one = zero + 1

    def chunk(c, carry):
        pltpu.sync_copy(nbr_hbm.at[pl.ds(tbase + c * _CT, _CT), :, :], nb_v)
        obase = c * (_CT * 8)

        def body(k, carry2):
            t_idx = 2 * k + t_off
            src = plsc.load_gather(nb_v, [t_idx, r_idx, zero])
            dst = plsc.load_gather(nb_v, [t_idx, r_idx, one])
            sl = pl.ds(obase + k * _L, _L)
            s_v[sl] = plsc.load_gather(z_v, [src])
            d_v[sl] = plsc.load_gather(z_v, [dst])
            return carry2

        lax.fori_loop(0, _CT * 8 // _L, body, 0, unroll=5)
        return carry

    lax.fori_loop(0, _NCHUNK, chunk, 0)
    pltpu.sync_copy(s_v, out_hbm.at[pl.ds(ebase, _EPW)])
    pltpu.sync_copy(d_v, out_hbm.at[pl.ds(N_EDGES + ebase, _EPW)])


@jax.jit
def _sc_index_gather(z_i32, nbr4):
    mesh = plsc.VectorSubcoreMesh(core_axis_name="c", subcore_axis_name="s")
    return pl.kernel(
        _sc_gather_body,
        out_type=jax.ShapeDtypeStruct((2 * N_EDGES,), jnp.int32),
        mesh=mesh,
        scratch_types=[
            pltpu.VMEM((N_NODES,), jnp.int32),
            pltpu.VMEM((_CT, 8, 2), jnp.int32),
            pltpu.VMEM((_EPW,), jnp.int32),
            pltpu.VMEM((_EPW,), jnp.int32),
        ],
        compiler_params=pltpu.CompilerParams(needs_layout_passes=False),
    )(z_i32, nbr4)


_BLK = 16000


_DOTT = (((0,), (0,)), ((), ()))


def _tc_dense_body(s_ref, d_ref, e_ref, t1_ref, t2_ref, wc_ref, b_ref, o_ref):
    s = s_ref[0]
    d = d_ref[0]
    rows = lax.broadcasted_iota(jnp.int32, (EMBED_DIM, _BLK), 0)
    ohT_s = (rows == s).astype(jnp.float32)
    ohT_d = (rows == d).astype(jnp.float32)
    acc = lax.dot_general(ohT_s, t1_ref[...], _DOTT,
                          preferred_element_type=jnp.float32)
    acc = acc + lax.dot_general(ohT_d, t2_ref[...], _DOTT,
                                preferred_element_type=jnp.float32)
    acc = acc + jnp.dot(e_ref[...], wc_ref[...], preferred_element_type=jnp.float32)
    acc = acc + b_ref[...]
    o_ref[...] = acc * (1.0 / (1.0 + jnp.exp(-acc)))


@jax.jit
def _tc_dense(s3, d3, e_rbf, t1, t2, w_c, b):
    grid = (N_EDGES // _BLK,)
    return pl.pallas_call(
        _tc_dense_body,
        grid=grid,
        in_specs=[
            pl.BlockSpec((1, 1, _BLK), lambda i: (i, 0, 0)),
            pl.BlockSpec((1, 1, _BLK), lambda i: (i, 0, 0)),
            pl.BlockSpec((_BLK, N_RBF), lambda i: (i, 0)),
            pl.BlockSpec((EMBED_DIM, EMBED_DIM), lambda i: (0, 0)),
            pl.BlockSpec((EMBED_DIM, EMBED_DIM), lambda i: (0, 0)),
            pl.BlockSpec((N_RBF, EMBED_DIM), lambda i: (0, 0)),
            pl.BlockSpec((1, EMBED_DIM), lambda i: (0, 0)),
        ],
        out_specs=pl.BlockSpec((_BLK, EMBED_DIM), lambda i: (i, 0)),
        out_shape=jax.ShapeDtypeStruct((N_EDGES, EMBED_DIM), jnp.float32),
        compiler_params=pltpu.CompilerParams(
            dimension_semantics=("arbitrary",),
        ),
    )(s3, d3, e_rbf, t1, t2, w_c, b)


def kernel(e_rbf, z, nbr_list, W_rbf, emb_table, W_edge, b_edge):
    W1 = W_edge[:EMBED_DIM]
    W2 = W_edge[EMBED_DIM : 2 * EMBED_DIM]
    W3 = W_edge[2 * EMBED_DIM :]
    t1 = jnp.zeros((EMBED_DIM, EMBED_DIM), jnp.float32).at[:100].set(emb_table @ W1)
    t2 = jnp.zeros((EMBED_DIM, EMBED_DIM), jnp.float32).at[:100].set(emb_table @ W2)
    w_c = W_rbf @ W3
    b = b_edge.reshape(1, EMBED_DIM)

    z_i32 = z.astype(jnp.int32)
    sd = _sc_index_gather(z_i32, nbr_list.reshape(N_EDGES // 8, 8, 2))
    nblk = N_EDGES // _BLK
    s3 = sd[:N_EDGES].reshape(nblk, 1, _BLK)
    d3 = sd[N_EDGES:].reshape(nblk, 1, _BLK)
    return _tc_dense(s3, d3, e_rbf, t1, t2, w_c, b)


# speedup vs baseline: 5.1818x; 1.0037x over previous
import functools

import jax
import jax.numpy as jnp
from jax import lax
from jax.experimental import pallas as pl
from jax.experimental.pallas import tpu as pltpu
from jax.experimental.pallas import tpu_sc as plsc

N_NODES = 10000
N_EDGES = 320000
N_RBF = 16
EMBED_DIM = 128


_NC, _NS, _L = 2, 16, 16
_NW = _NC * _NS
_EPW = N_EDGES // _NW


_TPW = _EPW // 8
_CT = 50
_NCHUNK = _TPW // _CT


def _sc_gather_body(z_hbm, nbr_hbm, out_hbm, z_v, nb_v, s_v, d_v):
    wid = lax.axis_index("s") * _NC + lax.axis_index("c")
    ebase = wid * _EPW
    tbase = wid * _TPW
    pltpu.sync_copy(z_hbm, z_v)
    lane = lax.iota(jnp.int32, _L)
    t_off = lax.shift_right_logical(lane, 3)
    r_idx = lane & 7
    zero = jnp.zeros((_L,), jnp.int32)
    one = zero + 1

    def chunk(c, carry):
        pltpu.sync_copy(nbr_hbm.at[pl.ds(tbase + c * _CT, _CT), :, :], nb_v)
        obase = c * (_CT * 8)

        def body(k, carry2):
            t_idx = 2 * k + t_off
            src = plsc.load_gather(nb_v, [t_idx, r_idx, zero])
            dst = plsc.load_gather(nb_v, [t_idx, r_idx, one])
            sl = pl.ds(obase + k * _L, _L)
            s_v[sl] = plsc.load_gather(z_v, [src])
            d_v[sl] = plsc.load_gather(z_v, [dst])
            return carry2

        lax.fori_loop(0, _CT * 8 // _L, body, 0, unroll=5)
        return carry

    lax.fori_loop(0, _NCHUNK, chunk, 0)
    pltpu.sync_copy(s_v, out_hbm.at[pl.ds(ebase, _EPW)])
    pltpu.sync_copy(d_v, out_hbm.at[pl.ds(N_EDGES + ebase, _EPW)])


@jax.jit
def _sc_index_gather(z_i32, nbr4):
    mesh = plsc.VectorSubcoreMesh(core_axis_name="c", subcore_axis_name="s")
    return pl.kernel(
        _sc_gather_body,
        out_type=jax.ShapeDtypeStruct((2 * N_EDGES,), jnp.int32),
        mesh=mesh,
        scratch_types=[
            pltpu.VMEM((N_NODES,), jnp.int32),
            pltpu.VMEM((_CT, 8, 2), jnp.int32),
            pltpu.VMEM((_EPW,), jnp.int32),
            pltpu.VMEM((_EPW,), jnp.int32),
        ],
        compiler_params=pltpu.CompilerParams(needs_layout_passes=False),
    )(z_i32, nbr4)


_BLK = 16000


_DOTT = (((0,), (0,)), ((), ()))


def _tc_dense_body(sd_ref, e_ref, t1_ref, t2_ref, wc_ref, b_ref, o_ref):
    s = sd_ref[0:1, :]
    d = sd_ref[1:2, :]
    rows = lax.broadcasted_iota(jnp.int32, (EMBED_DIM, _BLK), 0)
    ohT_s = (rows == s).astype(jnp.float32)
    ohT_d = (rows == d).astype(jnp.float32)
    acc = lax.dot_general(ohT_s, t1_ref[...], _DOTT,
                          preferred_element_type=jnp.float32)
    acc = acc + lax.dot_general(ohT_d, t2_ref[...], _DOTT,
                                preferred_element_type=jnp.float32)
    acc = acc + jnp.dot(e_ref[...], wc_ref[...], preferred_element_type=jnp.float32)
    acc = acc + b_ref[...]
    o_ref[...] = acc * (1.0 / (1.0 + jnp.exp(-acc)))


@jax.jit
def _tc_dense(sd2, e_rbf, t1, t2, w_c, b):
    grid = (N_EDGES // _BLK,)
    return pl.pallas_call(
        _tc_dense_body,
        grid=grid,
        in_specs=[
            pl.BlockSpec((2, _BLK), lambda i: (0, i)),
            pl.BlockSpec((_BLK, N_RBF), lambda i: (i, 0)),
            pl.BlockSpec((EMBED_DIM, EMBED_DIM), lambda i: (0, 0)),
            pl.BlockSpec((EMBED_DIM, EMBED_DIM), lambda i: (0, 0)),
            pl.BlockSpec((N_RBF, EMBED_DIM), lambda i: (0, 0)),
            pl.BlockSpec((1, EMBED_DIM), lambda i: (0, 0)),
        ],
        out_specs=pl.BlockSpec((_BLK, EMBED_DIM), lambda i: (i, 0)),
        out_shape=jax.ShapeDtypeStruct((N_EDGES, EMBED_DIM), jnp.float32),
        compiler_params=pltpu.CompilerParams(
            dimension_semantics=("arbitrary",),
        ),
    )(sd2, e_rbf, t1, t2, w_c, b)


def kernel(e_rbf, z, nbr_list, W_rbf, emb_table, W_edge, b_edge):
    W1 = W_edge[:EMBED_DIM]
    W2 = W_edge[EMBED_DIM : 2 * EMBED_DIM]
    W3 = W_edge[2 * EMBED_DIM :]
    t1 = jnp.zeros((EMBED_DIM, EMBED_DIM), jnp.float32).at[:100].set(emb_table @ W1)
    t2 = jnp.zeros((EMBED_DIM, EMBED_DIM), jnp.float32).at[:100].set(emb_table @ W2)
    w_c = W_rbf @ W3
    b = b_edge.reshape(1, EMBED_DIM)

    z_i32 = z.astype(jnp.int32)
    sd = _sc_index_gather(z_i32, nbr_list.reshape(N_EDGES // 8, 8, 2))
    return _tc_dense(sd.reshape(2, N_EDGES), e_rbf, t1, t2, w_c, b)


# speedup vs baseline: 5.1839x; 1.0004x over previous
import functools

import jax
import jax.numpy as jnp
from jax import lax
from jax.experimental import pallas as pl
from jax.experimental.pallas import tpu as pltpu
from jax.experimental.pallas import tpu_sc as plsc

N_NODES = 10000
N_EDGES = 320000
N_RBF = 16
EMBED_DIM = 128


_NC, _NS, _L = 2, 16, 16
_NW = _NC * _NS
_EPW = N_EDGES // _NW


_TPW = _EPW // 8
_CT = 50
_NCHUNK = _TPW // _CT


def _sc_gather_body(z_hbm, nbr_hbm, out_hbm, z_v, nb_v, s_v, d_v, dsem):
    wid = lax.axis_index("s") * _NC + lax.axis_index("c")
    ebase = wid * _EPW
    tbase = wid * _TPW
    pltpu.sync_copy(z_hbm, z_v)
    lane = lax.iota(jnp.int32, _L)
    t_off = lax.shift_right_logical(lane, 3)
    r_idx = lane & 7
    zero = jnp.zeros((_L,), jnp.int32)
    one = zero + 1

    def chunk(c, carry):
        pltpu.async_copy(
            nbr_hbm.at[pl.ds(tbase + c * _CT, _CT), :, :], nb_v, dsem).wait()
        obase = c * (_CT * 8)

        def body(k, carry2):
            t_idx = 2 * k + t_off
            src = plsc.load_gather(nb_v, [t_idx, r_idx, zero])
            dst = plsc.load_gather(nb_v, [t_idx, r_idx, one])
            sl = pl.ds(obase + k * _L, _L)
            s_v[sl] = plsc.load_gather(z_v, [src])
            d_v[sl] = plsc.load_gather(z_v, [dst])
            return carry2

        lax.fori_loop(0, _CT * 8 // _L, body, 0, unroll=5)
        return carry

    lax.fori_loop(0, _NCHUNK, chunk, 0)
    pltpu.sync_copy(s_v, out_hbm.at[pl.ds(ebase, _EPW)])
    pltpu.sync_copy(d_v, out_hbm.at[pl.ds(N_EDGES + ebase, _EPW)])


@jax.jit
def _sc_index_gather(z_i32, nbr4):
    mesh = plsc.VectorSubcoreMesh(core_axis_name="c", subcore_axis_name="s")
    return pl.kernel(
        _sc_gather_body,
        out_type=jax.ShapeDtypeStruct((2 * N_EDGES,), jnp.int32),
        mesh=mesh,
        scratch_types=[
            pltpu.VMEM((N_NODES,), jnp.int32),
            pltpu.VMEM((_CT, 8, 2), jnp.int32),
            pltpu.VMEM((_EPW,), jnp.int32),
            pltpu.VMEM((_EPW,), jnp.int32),
            pltpu.SemaphoreType.DMA,
        ],
        compiler_params=pltpu.CompilerParams(needs_layout_passes=False),
    )(z_i32, nbr4)


_BLK = 16000


_DOTT = (((0,), (0,)), ((), ()))


def _tc_dense_body(sd_ref, e_ref, t1_ref, t2_ref, wc_ref, b_ref, o_ref):
    s = sd_ref[0:1, :]
    d = sd_ref[1:2, :]
    rows = lax.broadcasted_iota(jnp.int32, (EMBED_DIM, _BLK), 0)
    ohT_s = (rows == s).astype(jnp.float32)
    ohT_d = (rows == d).astype(jnp.float32)
    acc = lax.dot_general(ohT_s, t1_ref[...], _DOTT,
                          preferred_element_type=jnp.float32)
    acc = acc + lax.dot_general(ohT_d, t2_ref[...], _DOTT,
                                preferred_element_type=jnp.float32)
    acc = acc + jnp.dot(e_ref[...], wc_ref[...], preferred_element_type=jnp.float32)
    acc = acc + b_ref[...]
    o_ref[...] = acc * (1.0 / (1.0 + jnp.exp(-acc)))


@jax.jit
def _tc_dense(sd2, e_rbf, t1, t2, w_c, b):
    grid = (N_EDGES // _BLK,)
    return pl.pallas_call(
        _tc_dense_body,
        grid=grid,
        in_specs=[
            pl.BlockSpec((2, _BLK), lambda i: (0, i)),
            pl.BlockSpec((_BLK, N_RBF), lambda i: (i, 0)),
            pl.BlockSpec((EMBED_DIM, EMBED_DIM), lambda i: (0, 0)),
            pl.BlockSpec((EMBED_DIM, EMBED_DIM), lambda i: (0, 0)),
            pl.BlockSpec((N_RBF, EMBED_DIM), lambda i: (0, 0)),
            pl.BlockSpec((1, EMBED_DIM), lambda i: (0, 0)),
        ],
        out_specs=pl.BlockSpec((_BLK, EMBED_DIM), lambda i: (i, 0)),
        out_shape=jax.ShapeDtypeStruct((N_EDGES, EMBED_DIM), jnp.float32),
        compiler_params=pltpu.CompilerParams(
            dimension_semantics=("arbitrary",),
        ),
    )(sd2, e_rbf, t1, t2, w_c, b)


def kernel(e_rbf, z, nbr_list, W_rbf, emb_table, W_edge, b_edge):
    W1 = W_edge[:EMBED_DIM]
    W2 = W_edge[EMBED_DIM : 2 * EMBED_DIM]
    W3 = W_edge[2 * EMBED_DIM :]
    t1 = jnp.zeros((EMBED_DIM, EMBED_DIM), jnp.float32).at[:100].set(emb_table @ W1)
    t2 = jnp.zeros((EMBED_DIM, EMBED_DIM), jnp.float32).at[:100].set(emb_table @ W2)
    w_c = W_rbf @ W3
    b = b_edge.reshape(1, EMBED_DIM)

    z_i32 = z.astype(jnp.int32)
    sd = _sc_index_gather(z_i32, nbr_list.reshape(N_EDGES // 8, 8, 2))
    return _tc_dense(sd.reshape(2, N_EDGES), e_rbf, t1, t2, w_c, b)
